# baseline (speedup 1.0000x reference)
import functools

import jax
import jax.numpy as jnp
from jax import lax
from jax.experimental import pallas as pl
from jax.experimental.pallas import tpu as pltpu
from jax.experimental.pallas import tpu_sc as plsc

B = 1024
CTX = 20
D = 64
V = 100000

NC = 2
NS = 16
NW = NC * NS
BPW = B // NW
IPW = BPW * CTX
CHUNK = 128
NCHUNK = IPW // CHUNK


def _pool_sc(xf, emb):
    mesh = plsc.VectorSubcoreMesh(
        core_axis_name="c", subcore_axis_name="s", num_cores=NC, num_subcores=NS
    )

    @functools.partial(
        pl.kernel,
        mesh=mesh,
        out_type=jax.ShapeDtypeStruct((B, D), jnp.float32),
        scratch_types=[
            pltpu.VMEM((IPW,), jnp.int32),
            pltpu.VMEM((IPW, D), jnp.float32),
            pltpu.VMEM((BPW, D), jnp.float32),
            pltpu.SemaphoreType.DMA,
        ],
        compiler_params=pltpu.CompilerParams(use_tc_tiling_on_sc=False),
    )
    def pool(xf_hbm, emb_hbm, out_hbm, idx_v, rows_v, m_v, sem):
        wid = lax.axis_index("s") * NC + lax.axis_index("c")
        base = wid * IPW
        pltpu.sync_copy(xf_hbm.at[pl.ds(base, IPW)], idx_v)
        copies = [
            pltpu.async_copy(
                emb_hbm.at[idx_v.at[pl.ds(j * CHUNK, CHUNK)]],
                rows_v.at[pl.ds(j * CHUNK, CHUNK)],
                sem,
            )
            for j in range(NCHUNK)
        ]
        for c in copies:
            c.wait()

        def body(i, carry):
            for d in range(D // 16):
                acc = rows_v[i * CTX, pl.ds(d * 16, 16)]
                for t in range(1, CTX):
                    acc = acc + rows_v[i * CTX + t, pl.ds(d * 16, 16)]
                m_v[i, pl.ds(d * 16, 16)] = acc * (1.0 / CTX)
            return carry

        lax.fori_loop(0, BPW, body, 0)
        pltpu.sync_copy(m_v, out_hbm.at[pl.ds(wid * BPW, BPW)])

    return pool(xf, emb)


VB = 2048


def _matmul_tc(m, W, b2):
    def mmk(m_ref, w_ref, b_ref, o_ref):
        o_ref[...] = (
            lax.dot_general(
                m_ref[...],
                w_ref[...],
                (((1,), (1,)), ((), ())),
                preferred_element_type=jnp.float32,
            )
            + b_ref[...]
        )

    return pl.pallas_call(
        mmk,
        grid=(pl.cdiv(V, VB),),
        in_specs=[
            pl.BlockSpec((B, D), lambda j: (0, 0)),
            pl.BlockSpec((VB, D), lambda j: (j, 0)),
            pl.BlockSpec((1, VB), lambda j: (0, j)),
        ],
        out_specs=pl.BlockSpec((B, VB), lambda j: (0, j)),
        out_shape=jax.ShapeDtypeStruct((B, V), jnp.float32),
    )(m, W, b2)


def kernel(x, emb, W, b):
    xf = x.reshape(-1)
    m = _pool_sc(xf, emb)
    return _matmul_tc(m, W, b.reshape(1, V))


# speedup vs baseline: 1.9090x; 1.9090x over previous
import functools

import jax
import jax.numpy as jnp
from jax import lax
from jax.experimental import pallas as pl
from jax.experimental.pallas import tpu as pltpu
from jax.experimental.pallas import tpu_sc as plsc

B = 1024
CTX = 20
D = 64
V = 100000

NC = 2
NS = 16
NW = NC * NS
BPW = B // NW
IPW = BPW * CTX
CHUNK = 128
NCHUNK = IPW // CHUNK


def _pool_sc(xf, emb):
    mesh = plsc.VectorSubcoreMesh(
        core_axis_name="c", subcore_axis_name="s", num_cores=NC, num_subcores=NS
    )

    @functools.partial(
        pl.kernel,
        mesh=mesh,
        out_type=jax.ShapeDtypeStruct((B, D), jnp.float32),
        scratch_types=[
            pltpu.VMEM((IPW,), jnp.int32),
            pltpu.VMEM((IPW, D), jnp.float32),
            pltpu.VMEM((BPW, D), jnp.float32),
            pltpu.SemaphoreType.DMA,
        ],
        compiler_params=pltpu.CompilerParams(use_tc_tiling_on_sc=False),
    )
    def pool(xf_hbm, emb_hbm, out_hbm, idx_v, rows_v, m_v, sem):
        wid = lax.axis_index("s") * NC + lax.axis_index("c")
        base = wid * IPW
        pltpu.sync_copy(xf_hbm.at[pl.ds(base, IPW)], idx_v)
        copies = [
            pltpu.async_copy(
                emb_hbm.at[idx_v.at[pl.ds(j * CHUNK, CHUNK)]],
                rows_v.at[pl.ds(j * CHUNK, CHUNK)],
                sem,
            )
            for j in range(NCHUNK)
        ]
        for c in copies:
            c.wait()

        def body(i, carry):
            for d in range(D // 16):
                acc = rows_v[i * CTX, pl.ds(d * 16, 16)]
                for t in range(1, CTX):
                    acc = acc + rows_v[i * CTX + t, pl.ds(d * 16, 16)]
                m_v[i, pl.ds(d * 16, 16)] = acc * (1.0 / CTX)
            return carry

        lax.fori_loop(0, BPW, body, 0)
        pltpu.sync_copy(m_v, out_hbm.at[pl.ds(wid * BPW, BPW)])

    return pool(xf, emb)


VB = 2048


def _matmul_tc(m, W, b2):
    def mmk(w_ref, m_ref, b_ref, o_ref):
        o_ref[...] = (
            lax.dot_general(
                w_ref[...],
                m_ref[...],
                (((1,), (1,)), ((), ())),
                preferred_element_type=jnp.float32,
            )
            + b_ref[...]
        )

    return pl.pallas_call(
        mmk,
        grid=(pl.cdiv(V, VB),),
        in_specs=[
            pl.BlockSpec((VB, D), lambda j: (j, 0)),
            pl.BlockSpec((B, D), lambda j: (0, 0)),
            pl.BlockSpec((VB, 1), lambda j: (j, 0)),
        ],
        out_specs=pl.BlockSpec((VB, B), lambda j: (j, 0)),
        out_shape=jax.ShapeDtypeStruct((V, B), jnp.float32),
    )(W, m, b2)


def kernel(x, emb, W, b):
    xf = x.reshape(-1)
    m = _pool_sc(xf, emb)
    return _matmul_tc(m, W, b.reshape(V, 1)).T


# speedup vs baseline: 2.7640x; 1.4479x over previous
import functools

import jax
import jax.numpy as jnp
from jax import lax
from jax.experimental import pallas as pl
from jax.experimental.pallas import tpu as pltpu
from jax.experimental.pallas import tpu_sc as plsc

B = 1024
CTX = 20
D = 64
V = 100000

NC = 2
NS = 16
NW = NC * NS
BPW = B // NW
IPW = BPW * CTX
CHUNK = 128
NCHUNK = IPW // CHUNK


def _pool_sc(xf, emb):
    mesh = plsc.VectorSubcoreMesh(
        core_axis_name="c", subcore_axis_name="s", num_cores=NC, num_subcores=NS
    )

    @functools.partial(
        pl.kernel,
        mesh=mesh,
        out_type=jax.ShapeDtypeStruct((B, D), jnp.float32),
        scratch_types=[
            pltpu.VMEM((IPW,), jnp.int32),
            pltpu.VMEM((IPW, D), jnp.float32),
            pltpu.VMEM((BPW, D), jnp.float32),
            pltpu.SemaphoreType.DMA,
        ],
        compiler_params=pltpu.CompilerParams(use_tc_tiling_on_sc=False),
    )
    def pool(xf_hbm, emb_hbm, out_hbm, idx_v, rows_v, m_v, sem):
        wid = lax.axis_index("s") * NC + lax.axis_index("c")
        base = wid * IPW
        pltpu.sync_copy(xf_hbm.at[pl.ds(base, IPW)], idx_v)
        copies = [
            pltpu.async_copy(
                emb_hbm.at[idx_v.at[pl.ds(j * CHUNK, CHUNK)]],
                rows_v.at[pl.ds(j * CHUNK, CHUNK)],
                sem,
            )
            for j in range(NCHUNK)
        ]
        for c in copies:
            c.wait()

        def body(i, carry):
            for d in range(D // 16):
                acc = rows_v[i * CTX, pl.ds(d * 16, 16)]
                for t in range(1, CTX):
                    acc = acc + rows_v[i * CTX + t, pl.ds(d * 16, 16)]
                m_v[i, pl.ds(d * 16, 16)] = acc * (1.0 / CTX)
            return carry

        lax.fori_loop(0, BPW, body, 0)
        pltpu.sync_copy(m_v, out_hbm.at[pl.ds(wid * BPW, BPW)])

    return pool(xf, emb)


VB = 2048


def _matmul_tc(m, W, b2):
    def mmk(wt_ref, m_ref, b_ref, o_ref):
        o_ref[...] = (
            lax.dot_general(
                wt_ref[...],
                m_ref[...],
                (((0,), (1,)), ((), ())),
                preferred_element_type=jnp.float32,
            )
            + b_ref[...][:, None]
        )

    return pl.pallas_call(
        mmk,
        grid=(pl.cdiv(V, VB),),
        in_specs=[
            pl.BlockSpec((D, VB), lambda j: (0, j)),
            pl.BlockSpec((B, D), lambda j: (0, 0)),
            pl.BlockSpec((VB,), lambda j: (j,)),
        ],
        out_specs=pl.BlockSpec((VB, B), lambda j: (j, 0)),
        out_shape=jax.ShapeDtypeStruct((V, B), jnp.float32),
        compiler_params=pltpu.CompilerParams(fuse_transposed_lhs_in_matmul=True),
    )(W, m, b2)


def kernel(x, emb, W, b):
    xf = x.reshape(-1)
    m = _pool_sc(xf, emb)
    return _matmul_tc(m, W.T, b).T
